# baseline (speedup 1.0000x reference)
import functools

import jax
import jax.numpy as jnp
from jax import lax
from jax.experimental import pallas as pl
from jax.experimental.pallas import tpu as pltpu, tpu_sc as plsc

D_MODEL_ = 4096
N_EXPERTS_ = 64
TOP_K_ = 8
BT_ = 1024
NW_ = 32
NCHUNK_ = 2


def _gate_block(x_ref, w_ref, probs_ref, psum_ref, acc_ref):
    i = pl.program_id(0)
    nsteps = pl.num_programs(0)

    @pl.when(i == 0)
    def _init():
        acc_ref[...] = jnp.zeros_like(acc_ref)

    logits = jax.lax.dot_general(
        w_ref[...], x_ref[...],
        dimension_numbers=(((1,), (1,)), ((), ())),
        preferred_element_type=jnp.float32,
    )
    m = jnp.max(logits, axis=0, keepdims=True)
    e = jnp.exp(logits - m)
    s = jnp.sum(e, axis=0, keepdims=True)
    probs = e / s
    rows = probs_ref.shape[0]
    span = BT_ // rows
    for r in range(rows):
        probs_ref[r] = probs[:, r * span:(r + 1) * span]
    acc_ref[...] += jnp.sum(probs, axis=1, keepdims=True)

    @pl.when(i == nsteps - 1)
    def _finish():
        psum_ref[...] = acc_ref[...]


def _make_sc_topk(span):
    grp = span // 16

    def _sc_topk(probs_hbm, vals_hbm, idx_hbm, cnts_hbm, buf, vbuf, ibuf, cnt):
        wid = lax.axis_index("s") * 2 + lax.axis_index("c")
        pltpu.sync_copy(probs_hbm.at[wid], buf)

        for i in range(N_EXPERTS_ // 16):
            cnt[pl.ds(i * 16, 16)] = jnp.zeros((16,), jnp.float32)

        iota = lax.iota(jnp.int32, 16)
        ones = jnp.ones((16,), jnp.float32)

        def body(g, carry):
            base = g * 16
            work = []
            for e in range(N_EXPERTS_):
                v = buf[e, pl.ds(base, 16)]
                b = lax.bitcast_convert_type(v, jnp.int32)
                work.append((b & -64) | (63 - e))
            rows = (base + iota) * TOP_K_
            for j in range(TOP_K_):
                mx = work[0]
                for e in range(1, N_EXPERTS_):
                    mx = jnp.maximum(mx, work[e])
                idxv = 63 - (mx & 63)
                valv = lax.bitcast_convert_type(mx & -64, jnp.float32)
                flat = rows + j
                plsc.store_scatter(vbuf, [flat], valv)
                plsc.store_scatter(ibuf, [flat], idxv)
                plsc.addupdate_scatter(cnt, [idxv], ones)
                if j != TOP_K_ - 1:
                    work = [jnp.where(w == mx, -1, w) for w in work]
            return carry

        lax.fori_loop(0, grp, body, 0)

        pltpu.sync_copy(vbuf, vals_hbm.at[pl.ds(wid * (span * TOP_K_),
                                                span * TOP_K_)])
        pltpu.sync_copy(ibuf, idx_hbm.at[pl.ds(wid * (span * TOP_K_),
                                               span * TOP_K_)])
        pltpu.sync_copy(cnt, cnts_hbm.at[wid])

    return _sc_topk


@functools.partial(jax.jit, static_argnames=())
def kernel(x, W):
    B, T, D = x.shape
    n_tok = B * T
    x2 = x.reshape(n_tok, D)
    ctok = n_tok // NCHUNK_
    span = ctok // NW_
    rows_per_step = BT_ // span

    mesh = plsc.VectorSubcoreMesh(core_axis_name="c", subcore_axis_name="s")
    sc_fn = pl.kernel(
        _make_sc_topk(span),
        out_type=[
            jax.ShapeDtypeStruct((ctok * TOP_K_,), jnp.float32),
            jax.ShapeDtypeStruct((ctok * TOP_K_,), jnp.int32),
            jax.ShapeDtypeStruct((NW_, N_EXPERTS_), jnp.float32),
        ],
        mesh=mesh,
        compiler_params=pltpu.CompilerParams(needs_layout_passes=False),
        scratch_types=[
            pltpu.VMEM((N_EXPERTS_, span), jnp.float32),
            pltpu.VMEM((span * TOP_K_,), jnp.float32),
            pltpu.VMEM((span * TOP_K_,), jnp.int32),
            pltpu.VMEM((N_EXPERTS_,), jnp.float32),
        ],
    )

    tc_fn = pl.pallas_call(
        _gate_block,
        grid=(ctok // BT_,),
        in_specs=[
            pl.BlockSpec((BT_, D), lambda i: (i, 0)),
            pl.BlockSpec((N_EXPERTS_, D), lambda i: (0, 0)),
        ],
        out_specs=[
            pl.BlockSpec((rows_per_step, N_EXPERTS_, span),
                         lambda i: (i, 0, 0)),
            pl.BlockSpec((N_EXPERTS_, 1), lambda i: (0, 0)),
        ],
        out_shape=[
            jax.ShapeDtypeStruct((NW_, N_EXPERTS_, span), jnp.float32),
            jax.ShapeDtypeStruct((N_EXPERTS_, 1), jnp.float32),
        ],
        scratch_shapes=[pltpu.VMEM((N_EXPERTS_, 1), jnp.float32)],
    )

    vals_c, idx_c = [], []
    psum_tot = jnp.zeros((N_EXPERTS_,), jnp.float32)
    cnt_tot = jnp.zeros((N_EXPERTS_,), jnp.float32)
    for c in range(NCHUNK_):
        probsP, psum = tc_fn(x2[c * ctok:(c + 1) * ctok], W)
        vals, idx, cnts = sc_fn(probsP)
        vals_c.append(vals)
        idx_c.append(idx)
        psum_tot = psum_tot + psum[:, 0]
        cnt_tot = cnt_tot + cnts.sum(axis=0)

    scale = 1.0 / (float(n_tok) * float(TOP_K_) * float(n_tok))
    loss = jnp.sum(cnt_tot * psum_tot) * scale
    vals = jnp.concatenate(vals_c).reshape(B, T, TOP_K_)
    idx = jnp.concatenate(idx_c).reshape(B, T, TOP_K_)
    return (vals, idx, loss)


# speedup vs baseline: 2.1544x; 2.1544x over previous
import functools

import jax
import jax.numpy as jnp
from jax import lax
from jax.experimental import pallas as pl
from jax.experimental.pallas import tpu as pltpu, tpu_sc as plsc

D_MODEL_ = 4096
N_EXPERTS_ = 64
TOP_K_ = 8
BT_ = 1024
NW_ = 32
NCHUNK_ = 2


def _gate_block(x_ref, w_ref, probs_ref, psum_ref, acc_ref):
    i = pl.program_id(0)
    nsteps = pl.num_programs(0)

    @pl.when(i == 0)
    def _init():
        acc_ref[...] = jnp.zeros_like(acc_ref)

    logits = jax.lax.dot_general(
        w_ref[...], x_ref[...],
        dimension_numbers=(((1,), (1,)), ((), ())),
        preferred_element_type=jnp.float32,
    )
    m = jnp.max(logits, axis=0, keepdims=True)
    e = jnp.exp(logits - m)
    s = jnp.sum(e, axis=0, keepdims=True)
    probs = e / s
    rows = probs_ref.shape[0]
    span = BT_ // rows
    for r in range(rows):
        probs_ref[r] = probs[:, r * span:(r + 1) * span]
    acc_ref[...] += jnp.sum(probs, axis=1, keepdims=True)

    @pl.when(i == nsteps - 1)
    def _finish():
        psum_ref[...] = acc_ref[...]


def _make_sc_topk(span):
    grp = span // 16

    def _sc_topk(probs_hbm, vals_hbm, idx_hbm, cnts_hbm, buf, vbuf, ibuf, cnt):
        wid = lax.axis_index("s") * 2 + lax.axis_index("c")
        pltpu.sync_copy(probs_hbm.at[wid], buf)

        for i in range(N_EXPERTS_ // 16):
            cnt[pl.ds(i * 16, 16)] = jnp.zeros((16,), jnp.float32)

        iota = lax.iota(jnp.int32, 16)
        ones = jnp.ones((16,), jnp.float32)

        def body(g, carry):
            base = g * 16
            work = []
            for e in range(N_EXPERTS_):
                v = buf[e, pl.ds(base, 16)]
                b = lax.bitcast_convert_type(v, jnp.int32)
                work.append((b & -64) | (63 - e))
            rows = (base + iota) * TOP_K_
            for j in range(TOP_K_):
                mx = work[0]
                for e in range(1, N_EXPERTS_):
                    mx = jnp.maximum(mx, work[e])
                idxv = 63 - (mx & 63)
                valv = lax.bitcast_convert_type(mx & -64, jnp.float32)
                flat = rows + j
                plsc.store_scatter(vbuf, [flat], valv)
                plsc.store_scatter(ibuf, [flat], idxv)
                plsc.addupdate_scatter(cnt, [idxv], ones)
                if j != TOP_K_ - 1:
                    work = [jnp.where(w == mx, -1, w) for w in work]
            return carry

        lax.fori_loop(0, grp, body, 0)

        pltpu.sync_copy(vbuf, vals_hbm.at[pl.ds(wid * (span * TOP_K_),
                                                span * TOP_K_)])
        pltpu.sync_copy(ibuf, idx_hbm.at[pl.ds(wid * (span * TOP_K_),
                                               span * TOP_K_)])
        pltpu.sync_copy(cnt, cnts_hbm.at[wid])

    return _sc_topk


@functools.partial(jax.jit, static_argnames=())
def kernel(x, W):
    B, T, D = x.shape
    n_tok = B * T
    x2 = x.reshape(n_tok, D)
    ctok = n_tok // NCHUNK_
    span = ctok // NW_
    rows_per_step = BT_ // span

    mesh = plsc.VectorSubcoreMesh(core_axis_name="c", subcore_axis_name="s")
    sc_fn = pl.kernel(
        _make_sc_topk(span),
        out_type=[
            jax.ShapeDtypeStruct((ctok * TOP_K_,), jnp.float32),
            jax.ShapeDtypeStruct((ctok * TOP_K_,), jnp.int32),
            jax.ShapeDtypeStruct((NW_, N_EXPERTS_), jnp.float32),
        ],
        mesh=mesh,
        compiler_params=pltpu.CompilerParams(needs_layout_passes=False),
        scratch_types=[
            pltpu.VMEM((N_EXPERTS_, span), jnp.float32),
            pltpu.VMEM((span * TOP_K_,), jnp.float32),
            pltpu.VMEM((span * TOP_K_,), jnp.int32),
            pltpu.VMEM((N_EXPERTS_,), jnp.float32),
        ],
    )

    def tc_fn(xa, Wa, c):
        steps = ctok // BT_
        return pl.pallas_call(
            _gate_block,
            grid=(steps,),
            in_specs=[
                pl.BlockSpec((BT_, D), lambda i, c=c: (i + c * steps, 0)),
                pl.BlockSpec((N_EXPERTS_, D), lambda i: (0, 0)),
            ],
            out_specs=[
                pl.BlockSpec((rows_per_step, N_EXPERTS_, span),
                             lambda i: (i, 0, 0)),
                pl.BlockSpec((N_EXPERTS_, 1), lambda i: (0, 0)),
            ],
            out_shape=[
                jax.ShapeDtypeStruct((NW_, N_EXPERTS_, span), jnp.float32),
                jax.ShapeDtypeStruct((N_EXPERTS_, 1), jnp.float32),
            ],
            scratch_shapes=[pltpu.VMEM((N_EXPERTS_, 1), jnp.float32)],
        )(xa, Wa)

    vals_c, idx_c = [], []
    psum_tot = jnp.zeros((N_EXPERTS_,), jnp.float32)
    cnt_tot = jnp.zeros((N_EXPERTS_,), jnp.float32)
    for c in range(NCHUNK_):
        probsP, psum = tc_fn(x2, W, c)
        vals, idx, cnts = sc_fn(probsP)
        vals_c.append(vals)
        idx_c.append(idx)
        psum_tot = psum_tot + psum[:, 0]
        cnt_tot = cnt_tot + cnts.sum(axis=0)

    scale = 1.0 / (float(n_tok) * float(TOP_K_) * float(n_tok))
    loss = jnp.sum(cnt_tot * psum_tot) * scale
    vals = jnp.concatenate(vals_c).reshape(B, T, TOP_K_)
    idx = jnp.concatenate(idx_c).reshape(B, T, TOP_K_)
    return (vals, idx, loss)


# speedup vs baseline: 3.0157x; 1.3998x over previous
import functools

import jax
import jax.numpy as jnp
from jax.experimental import pallas as pl
from jax.experimental.pallas import tpu as pltpu

D_MODEL_ = 4096
N_EXPERTS_ = 64
TOP_K_ = 8
BT_ = 1024


def _router_block(x_ref, w_ref, vals_ref, idx_ref, loss_ref, acc_ref):
    i = pl.program_id(0)
    nsteps = pl.num_programs(0)

    @pl.when(i == 0)
    def _init():
        acc_ref[...] = jnp.zeros_like(acc_ref)

    logits = jax.lax.dot_general(
        w_ref[...], x_ref[...],
        dimension_numbers=(((1,), (1,)), ((), ())),
        preferred_element_type=jnp.float32,
    )

    m = jnp.max(logits, axis=0, keepdims=True)
    e = jnp.exp(logits - m)
    s = jnp.sum(e, axis=0, keepdims=True)
    probs = e / s

    iota_e = jax.lax.broadcasted_iota(jnp.int32, probs.shape, 0)
    work = probs
    vals_rows = []
    idx_rows = []
    for _ in range(TOP_K_):
        mx = jnp.max(work, axis=0, keepdims=True)
        cand = jnp.where(work == mx, iota_e, N_EXPERTS_)
        sel = jnp.min(cand, axis=0, keepdims=True)
        vals_rows.append(mx)
        idx_rows.append(sel)
        work = jnp.where(iota_e == sel, -1.0, work)

    vals8 = jnp.concatenate(vals_rows, axis=0)
    idx8 = jnp.concatenate(idx_rows, axis=0)
    vals_ref[...] = vals8.T
    idx_ref[...] = idx8.T

    sel_mask = (work < 0.0).astype(jnp.float32)
    cnt_part = jnp.sum(sel_mask, axis=1, keepdims=True)
    p_part = jnp.sum(probs, axis=1, keepdims=True)
    acc_ref[:, 0:1] += cnt_part
    acc_ref[:, 1:2] += p_part

    @pl.when(i == nsteps - 1)
    def _finish():
        n_tok = nsteps * BT_
        cnt = acc_ref[:, 0:1]
        ps = acc_ref[:, 1:2]
        scale = 1.0 / (float(n_tok) * float(TOP_K_) * float(n_tok))
        loss_ref[...] = (jnp.sum(cnt * ps) * scale).reshape(1, 1)


@functools.partial(jax.jit, static_argnames=())
def kernel(x, W):
    B, T, D = x.shape
    n_tok = B * T
    x2 = x.reshape(n_tok, D)
    grid = (n_tok // BT_,)
    vals, idx, loss = pl.pallas_call(
        _router_block,
        grid=grid,
        in_specs=[
            pl.BlockSpec((BT_, D), lambda i: (i, 0)),
            pl.BlockSpec((N_EXPERTS_, D), lambda i: (0, 0)),
        ],
        out_specs=[
            pl.BlockSpec((BT_, TOP_K_), lambda i: (i, 0)),
            pl.BlockSpec((BT_, TOP_K_), lambda i: (i, 0)),
            pl.BlockSpec((1, 1), lambda i: (0, 0)),
        ],
        out_shape=[
            jax.ShapeDtypeStruct((n_tok, TOP_K_), jnp.float32),
            jax.ShapeDtypeStruct((n_tok, TOP_K_), jnp.int32),
            jax.ShapeDtypeStruct((1, 1), jnp.float32),
        ],
        scratch_shapes=[pltpu.VMEM((N_EXPERTS_, 2), jnp.float32)],
    )(x2, W)
    return (vals.reshape(B, T, TOP_K_), idx.reshape(B, T, TOP_K_),
            loss.reshape(()))
